# Initial kernel scaffold; baseline (speedup 1.0000x reference)
import functools

import jax
import jax.numpy as jnp
from jax import lax
from jax.experimental import pallas as pl
from jax.experimental.pallas import tpu as pltpu
from jax.experimental.pallas import tpu_sc as plsc

_NC = 2
_NS = 16
_NW = _NC * _NS

_D = 32
_G = 128
_K = 10
_C = _K * _G


def _gather_body(idx_hbm, table_hbm, out_hbm, idx_v, rows_v, sem):
    wid = lax.axis_index("s") * _NC + lax.axis_index("c")
    nrows_w = idx_hbm.shape[0] // _NW
    nchunks = nrows_w // _K
    row0 = wid * nrows_w

    def body(i, carry):
        r = row0 + i * _K
        pltpu.sync_copy(idx_hbm.at[pl.ds(r, _K)], idx_v)
        copies = []
        for j in range(_K):
            copies.append(
                pltpu.async_copy(
                    table_hbm.at[idx_v.at[j]],
                    rows_v.at[pl.ds(j * _G, _G)],
                    sem,
                )
            )
        for cp in copies:
            cp.wait()
        pltpu.sync_copy(rows_v, out_hbm.at[pl.ds(r * _G, _C)])
        return carry

    lax.fori_loop(0, nchunks, body, 0)


def kernel(inputs, table):
    batch, hist = inputs.shape
    b_total = batch * hist
    idx2d = inputs.reshape(b_total // _G, _G).astype(jnp.int32)

    mesh = plsc.VectorSubcoreMesh(core_axis_name="c", subcore_axis_name="s")
    run = functools.partial(
        pl.kernel,
        mesh=mesh,
        out_type=jax.ShapeDtypeStruct((b_total, _D), jnp.float32),
        scratch_types=[
            pltpu.VMEM((_K, _G), jnp.int32),
            pltpu.VMEM((_C, _D), jnp.float32),
            pltpu.SemaphoreType.DMA,
        ],
    )(_gather_body)

    out = run(idx2d, table)
    return out.reshape(batch, hist, _D)


# speedup vs baseline: 1.0952x; 1.0952x over previous
import functools

import jax
import jax.numpy as jnp
from jax import lax
from jax.experimental import pallas as pl
from jax.experimental.pallas import tpu as pltpu
from jax.experimental.pallas import tpu_sc as plsc

_NC = 2
_NS = 16
_NW = _NC * _NS

_D = 32
_G = 128
_K = 8
_C = _K * _G


def _gather_body(idx_hbm, table_hbm, out_hbm, idx_v, rows_v, sem):
    wid = lax.axis_index("s") * _NC + lax.axis_index("c")
    nrows_w = idx_hbm.shape[0] // _NW
    nchunks = nrows_w // _K
    row0 = wid * nrows_w

    def body(i, carry):
        r = row0 + i * _K
        pltpu.sync_copy(idx_hbm.at[pl.ds(r, _K)], idx_v)
        copies = []
        for j in range(_K):
            copies.append(
                pltpu.async_copy(
                    table_hbm.at[idx_v.at[j]],
                    rows_v.at[pl.ds(j * _G, _G)],
                    sem,
                )
            )
        for cp in copies:
            cp.wait()
        pltpu.sync_copy(rows_v, out_hbm.at[pl.ds(r * _G, _C)])
        return carry

    lax.fori_loop(0, nchunks, body, 0)


def kernel(inputs, table):
    batch, hist = inputs.shape
    b_total = batch * hist
    idx2d = inputs.reshape(b_total // _G, _G).astype(jnp.int32)

    mesh = plsc.VectorSubcoreMesh(core_axis_name="c", subcore_axis_name="s")
    run = functools.partial(
        pl.kernel,
        mesh=mesh,
        compiler_params=pltpu.CompilerParams(use_tc_tiling_on_sc=False),
        out_type=jax.ShapeDtypeStruct((b_total, _D), jnp.float32),
        scratch_types=[
            pltpu.VMEM((_K, _G), jnp.int32),
            pltpu.VMEM((_C, _D), jnp.float32),
            pltpu.SemaphoreType.DMA,
        ],
    )(_gather_body)

    out = run(idx2d, table)
    return out.reshape(batch, hist, _D)


# speedup vs baseline: 1.1141x; 1.0172x over previous
import functools

import jax
import jax.numpy as jnp
from jax import lax
from jax.experimental import pallas as pl
from jax.experimental.pallas import tpu as pltpu
from jax.experimental.pallas import tpu_sc as plsc

_NC = 2
_NS = 16
_NW = _NC * _NS

_D = 32
_G = 128
_K = 10
_C = _K * _G


def _gather_body(idx_hbm, table_hbm, out_hbm, idx_v, rows0, rows1,
                 sg0, sg1, so0, so1):
    wid = lax.axis_index("s") * _NC + lax.axis_index("c")
    nrows_w = idx_hbm.shape[0] // _NW
    nchunks = nrows_w // _K
    row0 = wid * nrows_w
    out0 = row0 * _G

    pltpu.sync_copy(idx_hbm.at[pl.ds(row0, nrows_w)], idx_v)

    rows = (rows0, rows1)
    sg = (sg0, sg1)
    so = (so0, so1)

    def issue_gathers(i, p):
        for j in range(_K):
            pltpu.async_copy(
                table_hbm.at[idx_v.at[i * _K + j]],
                rows[p].at[pl.ds(j * _G, _G)],
                sg[p],
            )

    def drain_gathers(p):
        pltpu.make_async_copy(out_hbm.at[pl.ds(0, _C)], rows[p], sg[p]).wait()

    def issue_store(i, p):
        pltpu.async_copy(rows[p], out_hbm.at[pl.ds(out0 + i * _C, _C)], so[p])

    def drain_store(p):
        pltpu.make_async_copy(rows[p], out_hbm.at[pl.ds(0, _C)], so[p]).wait()

    issue_gathers(0, 0)
    issue_gathers(1, 1)
    drain_gathers(0)
    issue_store(0, 0)

    def body(g, carry):
        i0 = 2 * g
        drain_store(0)
        issue_gathers(i0, 0)
        drain_gathers(1)
        issue_store(i0 - 1, 1)
        drain_store(1)
        issue_gathers(i0 + 1, 1)
        drain_gathers(0)
        issue_store(i0, 0)
        return carry

    lax.fori_loop(1, nchunks // 2, body, 0)

    drain_gathers(1)
    issue_store(nchunks - 1, 1)
    drain_store(0)
    drain_store(1)


def kernel(inputs, table):
    batch, hist = inputs.shape
    b_total = batch * hist
    idx2d = inputs.reshape(b_total // _G, _G).astype(jnp.int32)
    nrows_w = (b_total // _G) // _NW

    mesh = plsc.VectorSubcoreMesh(core_axis_name="c", subcore_axis_name="s")
    run = functools.partial(
        pl.kernel,
        mesh=mesh,
        compiler_params=pltpu.CompilerParams(use_tc_tiling_on_sc=False),
        out_type=jax.ShapeDtypeStruct((b_total, _D), jnp.float32),
        scratch_types=[
            pltpu.VMEM((nrows_w, _G), jnp.int32),
            pltpu.VMEM((_C, _D), jnp.float32),
            pltpu.VMEM((_C, _D), jnp.float32),
            pltpu.SemaphoreType.DMA,
            pltpu.SemaphoreType.DMA,
            pltpu.SemaphoreType.DMA,
            pltpu.SemaphoreType.DMA,
        ],
    )(_gather_body)

    out = run(idx2d, table)
    return out.reshape(batch, hist, _D)


# speedup vs baseline: 1.4532x; 1.3045x over previous
import functools

import jax
import jax.numpy as jnp
from jax import lax
from jax.experimental import pallas as pl
from jax.experimental.pallas import tpu as pltpu
from jax.experimental.pallas import tpu_sc as plsc

_NC = 2
_NS = 16
_NW = _NC * _NS

_D = 32
_G = 128


def _gather_body(idxT_hbm, table_hbm, out_hbm, idx_v, rows0, rows1,
                 tr0, tr1, sg0, sg1, so0, so1):
    H = idxT_hbm.shape[0]
    B = idxT_hbm.shape[1]
    bw = B // _NW
    jb = bw // _G
    nblk = H * jb
    wid = lax.axis_index("s") * _NC + lax.axis_index("c")
    bbase = wid * bw

    for j in range(jb):
        pltpu.sync_copy(idxT_hbm.at[:, pl.ds(bbase + j * _G, _G)],
                        idx_v.at[j])

    rows = (rows0, rows1)
    trs = (tr0, tr1)
    sg = (sg0, sg1)
    so = (so0, so1)
    iota = lax.iota(jnp.int32, 16)

    def issue_gather(i, p):
        pltpu.async_copy(table_hbm.at[idx_v.at[i % jb, i // jb]],
                         rows[p], sg[p])

    def drain_gather(p):
        pltpu.make_async_copy(table_hbm.at[pl.ds(0, _G)], rows[p],
                              sg[p]).wait()

    def transpose(p):
        r, t = rows[p], trs[p]

        def tbody(d, carry):
            c = jnp.full((16,), d, jnp.int32)
            for g in range(8):
                t[d, pl.ds(g * 16, 16)] = plsc.load_gather(
                    r, [iota + g * 16, c])
            return carry

        lax.fori_loop(0, _D, tbody, 0)

    def issue_store(i, p):
        pltpu.async_copy(
            trs[p],
            out_hbm.at[i // jb, :, pl.ds(bbase + (i % jb) * _G, _G)],
            so[p])

    def drain_store(p):
        pltpu.make_async_copy(trs[p], out_hbm.at[0, :, pl.ds(0, _G)],
                              so[p]).wait()

    issue_gather(0, 0)
    issue_gather(1, 1)
    drain_gather(0)
    transpose(0)
    issue_store(0, 0)

    def body(g, carry):
        i0 = 2 * g
        drain_store(0)
        issue_gather(i0, 0)
        drain_gather(1)
        transpose(1)
        issue_store(i0 - 1, 1)
        drain_store(1)
        issue_gather(i0 + 1, 1)
        drain_gather(0)
        transpose(0)
        issue_store(i0, 0)
        return carry

    lax.fori_loop(1, nblk // 2, body, 0)

    drain_gather(1)
    transpose(1)
    issue_store(nblk - 1, 1)
    drain_store(0)
    drain_store(1)


def kernel(inputs, table):
    batch, hist = inputs.shape
    bw = batch // _NW
    jb = bw // _G

    mesh = plsc.VectorSubcoreMesh(core_axis_name="c", subcore_axis_name="s")
    run = functools.partial(
        pl.kernel,
        mesh=mesh,
        compiler_params=pltpu.CompilerParams(
            use_tc_tiling_on_sc=False, needs_layout_passes=False),
        out_type=jax.ShapeDtypeStruct((hist, _D, batch), jnp.float32),
        scratch_types=[
            pltpu.VMEM((jb, hist, _G), jnp.int32),
            pltpu.VMEM((_G, _D), jnp.float32),
            pltpu.VMEM((_G, _D), jnp.float32),
            pltpu.VMEM((_D, _G), jnp.float32),
            pltpu.VMEM((_D, _G), jnp.float32),
            pltpu.SemaphoreType.DMA,
            pltpu.SemaphoreType.DMA,
            pltpu.SemaphoreType.DMA,
            pltpu.SemaphoreType.DMA,
        ],
    )(_gather_body)

    out_phys = run(inputs.T, table)
    return out_phys.transpose(2, 0, 1)


# speedup vs baseline: 2.2554x; 1.5520x over previous
import functools

import jax
import jax.numpy as jnp
from jax import lax
from jax.experimental import pallas as pl
from jax.experimental.pallas import tpu as pltpu
from jax.experimental.pallas import tpu_sc as plsc

_NC = 2
_NS = 16
_NW = _NC * _NS

_D = 32
_G = 128
_TP = 129
_NB = 4


def _gather_body(idxT_hbm, table_hbm, out_hbm, idx_v,
                 r0, r1, r2, r3, t0, t1, t2, t3,
                 sg0, sg1, sg2, sg3, so0, so1, so2, so3):
    H = idxT_hbm.shape[0]
    B = idxT_hbm.shape[1]
    bw = B // _NW
    jb = bw // _G
    nblk = H * jb
    wid = lax.axis_index("s") * _NC + lax.axis_index("c")
    bbase = wid * bw

    for j in range(jb):
        pltpu.sync_copy(idxT_hbm.at[:, pl.ds(bbase + j * _G, _G)],
                        idx_v.at[j])

    rows = (r0, r1, r2, r3)
    trs = (t0, t1, t2, t3)
    sg = (sg0, sg1, sg2, sg3)
    so = (so0, so1, so2, so3)
    iota = lax.iota(jnp.int32, 16)

    def issue_gather(i, p):
        pltpu.async_copy(table_hbm.at[idx_v.at[i % jb, i // jb]],
                         rows[p], sg[p])

    def drain_gather(p):
        pltpu.make_async_copy(table_hbm.at[pl.ds(0, _G)],
                              rows[p], sg[p]).wait()

    def transpose(p):
        r, t = rows[p], trs[p]
        iota_hi = iota + 16

        def tbody(b, carry):
            c = jnp.full((16,), b, jnp.int32)
            plsc.store_scatter(t, [iota, c], r[b, pl.ds(0, 16)])
            plsc.store_scatter(t, [iota_hi, c], r[b, pl.ds(16, 16)])
            return carry

        lax.fori_loop(0, _G, tbody, 0, unroll=8)

    def issue_store(i, p):
        pltpu.async_copy(
            trs[p].at[:, pl.ds(0, _G)],
            out_hbm.at[i // jb, :, pl.ds(bbase + (i % jb) * _G, _G)],
            so[p])

    def drain_store(p):
        pltpu.make_async_copy(trs[p].at[:, pl.ds(0, _G)],
                              out_hbm.at[0, :, pl.ds(0, _G)], so[p]).wait()

    def step(i, p, q, with_drain_store=True):
        issue_gather(i, p)
        drain_gather(q)
        if with_drain_store:
            drain_store(q)
        transpose(q)
        issue_store(i - 2, q)

    issue_gather(0, 0)
    issue_gather(1, 1)
    step(2, 2, 0, with_drain_store=False)
    step(3, 3, 1, with_drain_store=False)
    step(4, 0, 2, with_drain_store=False)
    step(5, 1, 3, with_drain_store=False)
    step(6, 2, 0)
    step(7, 3, 1)

    def body(g, carry):
        i0 = 4 * g
        step(i0, 0, 2)
        step(i0 + 1, 1, 3)
        step(i0 + 2, 2, 0)
        step(i0 + 3, 3, 1)
        return carry

    lax.fori_loop(2, nblk // _NB, body, 0)

    for i in (nblk, nblk + 1):
        q = (i - 2) % _NB
        drain_gather(q)
        drain_store(q)
        transpose(q)
        issue_store(i - 2, q)
    for p in range(_NB):
        drain_store(p)


def kernel(inputs, table):
    batch, hist = inputs.shape
    bw = batch // _NW
    jb = bw // _G

    mesh = plsc.VectorSubcoreMesh(core_axis_name="c", subcore_axis_name="s")
    run = functools.partial(
        pl.kernel,
        mesh=mesh,
        compiler_params=pltpu.CompilerParams(
            use_tc_tiling_on_sc=False, needs_layout_passes=False),
        out_type=jax.ShapeDtypeStruct((hist, _D, batch), jnp.float32),
        scratch_types=(
            [pltpu.VMEM((jb, hist, _G), jnp.int32)]
            + [pltpu.VMEM((_G, _D), jnp.float32) for _ in range(_NB)]
            + [pltpu.VMEM((_D, _TP), jnp.float32) for _ in range(_NB)]
            + [pltpu.SemaphoreType.DMA for _ in range(2 * _NB)]
        ),
    )(_gather_body)

    out_phys = run(inputs.T, table)
    return out_phys.transpose(2, 0, 1)
